# baseline (speedup 1.0000x reference)
import functools
import math

import jax
import jax.numpy as jnp
from jax import lax
from jax.experimental import pallas as pl
from jax.experimental.pallas import tpu as pltpu

_CUTOFF = 4.0
_GSTEP = 0.1
_LN2 = math.log(2.0)


def _ssp(x):
    return jnp.maximum(x, 0.0) + jnp.log1p(jnp.exp(-jnp.abs(x))) - _LN2


def _sigmoid(x):
    return 1.0 / (1.0 + jnp.exp(-x))


def _pick_block(n, candidates):
    for c in candidates:
        if n % c == 0:
            return c
    return n


def _msg_body(with_dst, hs_ref, hd_ref, d_ref, w1a_ref, w1b_ref, b1_ref,
              w2_ref, b2_ref, we1_ref, be1_ref, we2_ref, be2_ref, out_ref):
    f32 = jnp.float32
    pre = jnp.dot(hs_ref[...], w1a_ref[...], preferred_element_type=f32)
    if with_dst:
        pre = pre + jnp.dot(hd_ref[...], w1b_ref[...],
                            preferred_element_type=f32)
    pre = pre + b1_ref[...][None, :]
    core = jnp.dot(_ssp(pre), w2_ref[...], preferred_element_type=f32)
    core = core + b2_ref[...][None, :]

    d = d_ref[...]
    nmu = we1_ref.shape[0]
    mu = _GSTEP * lax.broadcasted_iota(
        jnp.int32, (d.shape[0], nmu), 1).astype(f32)
    gauss = jnp.exp(-((d - mu) ** 2) / (2.0 * _GSTEP * _GSTEP))
    gh = _ssp(jnp.dot(gauss, we1_ref[...], preferred_element_type=f32)
              + be1_ref[...][None, :])
    gate = jnp.dot(gh, we2_ref[...], preferred_element_type=f32)
    gate = gate + be2_ref[...][None, :]
    cut = 1.0 - _sigmoid(5.0 * (d - (_CUTOFF - 1.5)))
    out_ref[...] = core * (gate * cut)


def _messages(hs, hd, d, mp):
    E, H = hs.shape
    w1 = mp["node"]["w1"]
    w1a, w1b = w1[:H], w1[H:]
    eb = _pick_block(E, (2000, 1000, 800, 400, 200, 80, 40, 16, 8))
    grid = E // eb
    with_dst = hd is not None
    if not with_dst:
        hd = hs

    def full(a):
        return pl.BlockSpec(a.shape, lambda i: (0,) * a.ndim)

    row = pl.BlockSpec((eb, H), lambda i: (i, 0))
    in_specs = [row, row, pl.BlockSpec((eb, 1), lambda i: (i, 0)),
                full(w1a), full(w1b), full(mp["node"]["b1"]),
                full(mp["node"]["w2"]), full(mp["node"]["b2"]),
                full(mp["edge"]["w1"]), full(mp["edge"]["b1"]),
                full(mp["edge"]["w2"]), full(mp["edge"]["b2"])]
    out_h = mp["node"]["w2"].shape[1]
    return pl.pallas_call(
        functools.partial(_msg_body, with_dst),
        grid=(grid,),
        in_specs=in_specs,
        out_specs=pl.BlockSpec((eb, out_h), lambda i: (i, 0)),
        out_shape=jax.ShapeDtypeStruct((E, out_h), jnp.float32),
    )(hs, hd, d, w1a, w1b, mp["node"]["b1"], mp["node"]["w2"],
      mp["node"]["b2"], mp["edge"]["w1"], mp["edge"]["b1"],
      mp["edge"]["w2"], mp["edge"]["b2"])


def _atom_upd_body(h_ref, p0_ref, p1_ref, w1_ref, b1_ref, w2_ref, b2_ref,
                   out_ref):
    f32 = jnp.float32
    msum = p0_ref[...] + p1_ref[...]
    hid = _ssp(jnp.dot(msum, w1_ref[...], preferred_element_type=f32)
               + b1_ref[...][None, :])
    t = jnp.dot(hid, w2_ref[...], preferred_element_type=f32) \
        + b2_ref[...][None, :]
    out_ref[...] = h_ref[...] + t


def _atom_update(h, p0, p1, sp):
    N, H = h.shape
    nb = _pick_block(N, (2000, 1000, 800, 400, 200, 80, 40, 16, 8))

    def full(a):
        return pl.BlockSpec(a.shape, lambda i: (0,) * a.ndim)

    row = pl.BlockSpec((nb, H), lambda i: (i, 0))
    return pl.pallas_call(
        _atom_upd_body,
        grid=(N // nb,),
        in_specs=[row, row, row, full(sp["w1"]), full(sp["b1"]),
                  full(sp["w2"]), full(sp["b2"])],
        out_specs=row,
        out_shape=jax.ShapeDtypeStruct((N, H), jnp.float32),
    )(h, p0, p1, sp["w1"], sp["b1"], sp["w2"], sp["b2"])


def _probe_upd_body(ps_ref, p0_ref, p1_ref, gw1_ref, gb1_ref, gw2_ref,
                    gb2_ref, tw1_ref, tb1_ref, tw2_ref, tb2_ref, out_ref):
    f32 = jnp.float32
    ps = ps_ref[...]
    msum = p0_ref[...] + p1_ref[...]
    ghid = _ssp(jnp.dot(ps, gw1_ref[...], preferred_element_type=f32)
                + gb1_ref[...][None, :])
    g = _sigmoid(jnp.dot(ghid, gw2_ref[...], preferred_element_type=f32)
                 + gb2_ref[...][None, :])
    thid = _ssp(jnp.dot(msum, tw1_ref[...], preferred_element_type=f32)
                + tb1_ref[...][None, :])
    t = jnp.dot(thid, tw2_ref[...], preferred_element_type=f32) \
        + tb2_ref[...][None, :]
    out_ref[...] = ps * g + (1.0 - g) * t


def _probe_update(ps, p0, p1, gp, tp):
    N, H = ps.shape
    nb = _pick_block(N, (2000, 1000, 800, 400, 200, 80, 40, 16, 8))

    def full(a):
        return pl.BlockSpec(a.shape, lambda i: (0,) * a.ndim)

    row = pl.BlockSpec((nb, H), lambda i: (i, 0))
    return pl.pallas_call(
        _probe_upd_body,
        grid=(N // nb,),
        in_specs=[row, row, row,
                  full(gp["w1"]), full(gp["b1"]), full(gp["w2"]),
                  full(gp["b2"]),
                  full(tp["w1"]), full(tp["b1"]), full(tp["w2"]),
                  full(tp["b2"])],
        out_specs=row,
        out_shape=jax.ShapeDtypeStruct((N, H), jnp.float32),
    )(ps, p0, p1, gp["w1"], gp["b1"], gp["w2"], gp["b2"],
      tp["w1"], tp["b1"], tp["w2"], tp["b2"])


def _emb_body(idx_ref, emb_ref, out_ref):
    f32 = jnp.float32
    nel = emb_ref.shape[0]
    idx = idx_ref[...]
    iota = lax.broadcasted_iota(jnp.int32, (idx.shape[0], nel), 1)
    onehot = (idx == iota).astype(f32)
    out_ref[...] = jnp.dot(onehot, emb_ref[...], preferred_element_type=f32)


def _embed(nodes_flat, emb):
    N = nodes_flat.shape[0]
    H = emb.shape[1]
    nb = _pick_block(N, (2000, 1000, 800, 400, 200, 80, 40, 16, 8))
    idx2 = nodes_flat.reshape(N, 1).astype(jnp.int32)
    return pl.pallas_call(
        _emb_body,
        grid=(N // nb,),
        in_specs=[pl.BlockSpec((nb, 1), lambda i: (i, 0)),
                  pl.BlockSpec(emb.shape, lambda i: (0, 0))],
        out_specs=pl.BlockSpec((nb, H), lambda i: (i, 0)),
        out_shape=jax.ShapeDtypeStruct((N, H), jnp.float32),
    )(idx2, emb)


def _readout_body(ps_ref, w1_ref, b1_ref, w2_ref, b2_ref, out_ref):
    f32 = jnp.float32
    hid = _ssp(jnp.dot(ps_ref[...], w1_ref[...], preferred_element_type=f32)
               + b1_ref[...][None, :])
    out_ref[...] = jnp.dot(hid, w2_ref[...], preferred_element_type=f32) \
        + b2_ref[...][None, :]


def _readout(ps, rp):
    N, H = ps.shape
    nb = _pick_block(N, (2000, 1000, 800, 400, 200, 80, 40, 16, 8))

    def full(a):
        return pl.BlockSpec(a.shape, lambda i: (0,) * a.ndim)

    return pl.pallas_call(
        _readout_body,
        grid=(N // nb,),
        in_specs=[pl.BlockSpec((nb, H), lambda i: (i, 0)),
                  full(rp["w1"]), full(rp["b1"]), full(rp["w2"]),
                  full(rp["b2"])],
        out_specs=pl.BlockSpec((nb, 1), lambda i: (i, 0)),
        out_shape=jax.ShapeDtypeStruct((N, 1), jnp.float32),
    )(ps, rp["w1"], rp["b1"], rp["w2"], rp["b2"])


def _gather_rows(table, idx):
    return jnp.take(table, idx, axis=0)


def _scatter_add(messages, dst, n_rows):
    out = jnp.zeros((n_rows, messages.shape[1]), dtype=messages.dtype)
    return out.at[dst].add(messages)


_PROBES_PER = 2500


def kernel(nodes, num_nodes, atom_edges, atom_edges_features, num_atom_edges,
           probe_edges, probe_edges_features, num_probes, num_probe_edges,
           params):
    bsz, nodes_per = nodes.shape
    N = bsz * nodes_per
    E = bsz * atom_edges.shape[1]

    idx_dtype = jnp.int32
    node_off = (jnp.arange(bsz, dtype=idx_dtype) * nodes_per)
    edges = (atom_edges.astype(idx_dtype)
             + node_off[:, None, None]).reshape(E, 2)
    e_src = edges[:, 0]
    e_dst = edges[:, 1]
    d_atom = atom_edges_features.reshape(E, 1).astype(jnp.float32)

    pe = bsz * probe_edges.shape[1]
    probes_per = _PROBES_PER
    P = bsz * probes_per
    probe_off = (jnp.arange(bsz, dtype=idx_dtype) * probes_per)
    pedges = probe_edges.astype(idx_dtype) + jnp.stack(
        [node_off, probe_off], axis=1)[:, None, :]
    pedges = pedges.reshape(pe, 2)
    pe_src = pedges[:, 0]
    pe_dst = pedges[:, 1]
    d_probe = probe_edges_features.reshape(pe, 1).astype(jnp.float32)

    h = _embed(nodes.reshape(N), params["atom_emb"])
    atom_reps = []
    for p in params["atom_int"]:
        hs = _gather_rows(h, e_src)
        hd = _gather_rows(h, e_dst)
        msg = _messages(hs, hd, d_atom, p["msg"])
        msum = _scatter_add(msg, e_dst, N)
        zeros = jnp.zeros_like(msum)
        h = _atom_update(h, msum, zeros, p["st"])
        atom_reps.append(h)

    ps = jnp.zeros((P, h.shape[1]), dtype=jnp.float32)
    for i, (p, nod) in enumerate(zip(params["probe"], atom_reps)):
        hs = _gather_rows(nod, pe_src)
        if i == 0:
            msg = _messages(hs, None, d_probe, p["msg"])
        else:
            hd = _gather_rows(ps, pe_dst)
            msg = _messages(hs, hd, d_probe, p["msg"])
        msum = _scatter_add(msg, pe_dst, P)
        zeros = jnp.zeros_like(msum)
        ps = _probe_update(ps, msum, zeros, p["gate"], p["trans"])

    out = _readout(ps, params["readout"])
    return out.reshape(bsz, probes_per)


# speedup vs baseline: 2.0808x; 2.0808x over previous
import functools
import math

import jax
import jax.numpy as jnp
from jax import lax
from jax.experimental import pallas as pl
from jax.experimental.pallas import tpu as pltpu
from jax.experimental.pallas import tpu_sc as plsc

_SC_CORES = 2
_SC_SUBCORES = 16
_SC_WORKERS = _SC_CORES * _SC_SUBCORES


def _sc_mesh():
    return plsc.VectorSubcoreMesh(core_axis_name="c", subcore_axis_name="s",
                                  num_cores=_SC_CORES,
                                  num_subcores=_SC_SUBCORES)


def _chunk_of(rows):
    for c in range(128, 7, -8):
        if rows % c == 0:
            return c
    return rows

_CUTOFF = 4.0
_GSTEP = 0.1
_LN2 = math.log(2.0)


def _ssp(x):
    return jnp.maximum(x, 0.0) + jnp.log1p(jnp.exp(-jnp.abs(x))) - _LN2


def _sigmoid(x):
    return 1.0 / (1.0 + jnp.exp(-x))


def _pick_block(n, candidates):
    for c in candidates:
        if n % c == 0:
            return c
    return n


def _msg_body(with_dst, hs_ref, hd_ref, d_ref, w1a_ref, w1b_ref, b1_ref,
              w2_ref, b2_ref, we1_ref, be1_ref, we2_ref, be2_ref, out_ref):
    f32 = jnp.float32
    pre = jnp.dot(hs_ref[...], w1a_ref[...], preferred_element_type=f32)
    if with_dst:
        pre = pre + jnp.dot(hd_ref[...], w1b_ref[...],
                            preferred_element_type=f32)
    pre = pre + b1_ref[...][None, :]
    core = jnp.dot(_ssp(pre), w2_ref[...], preferred_element_type=f32)
    core = core + b2_ref[...][None, :]

    d = d_ref[...]
    nmu = we1_ref.shape[0]
    mu = _GSTEP * lax.broadcasted_iota(
        jnp.int32, (d.shape[0], nmu), 1).astype(f32)
    gauss = jnp.exp(-((d - mu) ** 2) / (2.0 * _GSTEP * _GSTEP))
    gh = _ssp(jnp.dot(gauss, we1_ref[...], preferred_element_type=f32)
              + be1_ref[...][None, :])
    gate = jnp.dot(gh, we2_ref[...], preferred_element_type=f32)
    gate = gate + be2_ref[...][None, :]
    cut = 1.0 - _sigmoid(5.0 * (d - (_CUTOFF - 1.5)))
    out_ref[...] = core * (gate * cut)


def _messages(E, hs, hs_base, hd, hd_base, d, mp):
    H = hs.shape[1]
    w1 = mp["node"]["w1"]
    w1a, w1b = w1[:H], w1[H:]
    eb = _pick_block(E, (2000, 1000, 800, 400, 200, 80, 40, 16, 8))
    grid = E // eb
    assert hs_base % eb == 0 and (hd_base % eb == 0)
    sb = hs_base // eb
    db = hd_base // eb
    with_dst = hd is not None
    if not with_dst:
        hd = hs

    def full(a):
        return pl.BlockSpec(a.shape, lambda i: (0,) * a.ndim)

    src_spec = pl.BlockSpec((eb, H), lambda i: (sb + i, 0))
    dst_spec = pl.BlockSpec((eb, H), lambda i: (db + i, 0))
    in_specs = [src_spec, dst_spec, pl.BlockSpec((eb, 1), lambda i: (i, 0)),
                full(w1a), full(w1b), full(mp["node"]["b1"]),
                full(mp["node"]["w2"]), full(mp["node"]["b2"]),
                full(mp["edge"]["w1"]), full(mp["edge"]["b1"]),
                full(mp["edge"]["w2"]), full(mp["edge"]["b2"])]
    out_h = mp["node"]["w2"].shape[1]
    return pl.pallas_call(
        functools.partial(_msg_body, with_dst),
        grid=(grid,),
        in_specs=in_specs,
        out_specs=pl.BlockSpec((eb, out_h), lambda i: (i, 0)),
        out_shape=jax.ShapeDtypeStruct((E, out_h), jnp.float32),
    )(hs, hd, d, w1a, w1b, mp["node"]["b1"], mp["node"]["w2"],
      mp["node"]["b2"], mp["edge"]["w1"], mp["edge"]["b1"],
      mp["edge"]["w2"], mp["edge"]["b2"])


def _atom_upd_body(h_ref, p0_ref, p1_ref, w1_ref, b1_ref, w2_ref, b2_ref,
                   out_ref):
    f32 = jnp.float32
    msum = p0_ref[0] + p1_ref[0]
    hid = _ssp(jnp.dot(msum, w1_ref[...], preferred_element_type=f32)
               + b1_ref[...][None, :])
    t = jnp.dot(hid, w2_ref[...], preferred_element_type=f32) \
        + b2_ref[...][None, :]
    out_ref[...] = h_ref[...] + t


def _atom_update(h, parts, sp):
    N, H = h.shape
    nb = 400

    def full(a):
        return pl.BlockSpec(a.shape, lambda i: (0,) * a.ndim)

    row = pl.BlockSpec((nb, H), lambda i: (i, 0))
    p0row = pl.BlockSpec((1, nb, H), lambda i: (0, i, 0))
    p1row = pl.BlockSpec((1, nb, H), lambda i: (1, i, 0))
    return pl.pallas_call(
        _atom_upd_body,
        grid=(N // nb,),
        in_specs=[row, p0row, p1row, full(sp["w1"]), full(sp["b1"]),
                  full(sp["w2"]), full(sp["b2"])],
        out_specs=row,
        out_shape=jax.ShapeDtypeStruct((N, H), jnp.float32),
    )(h, parts, parts, sp["w1"], sp["b1"], sp["w2"], sp["b2"])


def _probe_upd_body(ps_ref, p0_ref, p1_ref, gw1_ref, gb1_ref, gw2_ref,
                    gb2_ref, tw1_ref, tb1_ref, tw2_ref, tb2_ref, out_ref):
    f32 = jnp.float32
    ps = ps_ref[...]
    msum = p0_ref[0] + p1_ref[0]
    ghid = _ssp(jnp.dot(ps, gw1_ref[...], preferred_element_type=f32)
                + gb1_ref[...][None, :])
    g = _sigmoid(jnp.dot(ghid, gw2_ref[...], preferred_element_type=f32)
                 + gb2_ref[...][None, :])
    thid = _ssp(jnp.dot(msum, tw1_ref[...], preferred_element_type=f32)
                + tb1_ref[...][None, :])
    t = jnp.dot(thid, tw2_ref[...], preferred_element_type=f32) \
        + tb2_ref[...][None, :]
    out_ref[...] = ps * g + (1.0 - g) * t


def _probe_update(ps, parts, gp, tp):
    N, H = ps.shape
    nb = 400

    def full(a):
        return pl.BlockSpec(a.shape, lambda i: (0,) * a.ndim)

    row = pl.BlockSpec((nb, H), lambda i: (i, 0))
    p0row = pl.BlockSpec((1, nb, H), lambda i: (0, i, 0))
    p1row = pl.BlockSpec((1, nb, H), lambda i: (1, i, 0))
    return pl.pallas_call(
        _probe_upd_body,
        grid=(N // nb,),
        in_specs=[row, p0row, p1row,
                  full(gp["w1"]), full(gp["b1"]), full(gp["w2"]),
                  full(gp["b2"]),
                  full(tp["w1"]), full(tp["b1"]), full(tp["w2"]),
                  full(tp["b2"])],
        out_specs=row,
        out_shape=jax.ShapeDtypeStruct((N, H), jnp.float32),
    )(ps, parts, parts, gp["w1"], gp["b1"], gp["w2"], gp["b2"],
      tp["w1"], tp["b1"], tp["w2"], tp["b2"])


def _emb_body(idx_ref, emb_ref, out_ref):
    f32 = jnp.float32
    nel = emb_ref.shape[0]
    idx = idx_ref[...]
    iota = lax.broadcasted_iota(jnp.int32, (idx.shape[0], nel), 1)
    onehot = (idx == iota).astype(f32)
    out_ref[...] = jnp.dot(onehot, emb_ref[...], preferred_element_type=f32)


def _embed(nodes_flat, emb):
    N = nodes_flat.shape[0]
    H = emb.shape[1]
    nb = _pick_block(N, (2000, 1000, 800, 400, 200, 80, 40, 16, 8))
    idx2 = nodes_flat.reshape(N, 1).astype(jnp.int32)
    return pl.pallas_call(
        _emb_body,
        grid=(N // nb,),
        in_specs=[pl.BlockSpec((nb, 1), lambda i: (i, 0)),
                  pl.BlockSpec(emb.shape, lambda i: (0, 0))],
        out_specs=pl.BlockSpec((nb, H), lambda i: (i, 0)),
        out_shape=jax.ShapeDtypeStruct((N, H), jnp.float32),
    )(idx2, emb)


def _readout_body(ps_ref, w1_ref, b1_ref, w2_ref, b2_ref, out_ref):
    f32 = jnp.float32
    hid = _ssp(jnp.dot(ps_ref[...], w1_ref[...], preferred_element_type=f32)
               + b1_ref[...][None, :])
    out_ref[...] = jnp.dot(hid, w2_ref[...], preferred_element_type=f32) \
        + b2_ref[...][None, :]


def _readout(ps, rp):
    N, H = ps.shape
    nb = _pick_block(N, (2000, 1000, 800, 400, 200, 80, 40, 16, 8))

    def full(a):
        return pl.BlockSpec(a.shape, lambda i: (0,) * a.ndim)

    return pl.pallas_call(
        _readout_body,
        grid=(N // nb,),
        in_specs=[pl.BlockSpec((nb, H), lambda i: (i, 0)),
                  full(rp["w1"]), full(rp["b1"]), full(rp["w2"]),
                  full(rp["b2"])],
        out_specs=pl.BlockSpec((nb, 1), lambda i: (i, 0)),
        out_shape=jax.ShapeDtypeStruct((N, 1), jnp.float32),
    )(ps, rp["w1"], rp["b1"], rp["w2"], rp["b2"])


def _sc_gather(table, idx):
    E = idx.shape[0]
    H = table.shape[1]
    rpw = E // _SC_WORKERS
    ch = _chunk_of(rpw)
    nc = rpw // ch

    @functools.partial(
        pl.kernel,
        out_type=jax.ShapeDtypeStruct((E, H), jnp.float32),
        mesh=_sc_mesh(),
        scratch_types=[pltpu.VMEM((ch,), jnp.int32),
                       pltpu.VMEM((ch, H), jnp.float32),
                       pltpu.SemaphoreType.DMA],
    )
    def gk(table_hbm, idx_hbm, out_hbm, idx_v, rows_v, sem):
        wid = lax.axis_index("s") * _SC_CORES + lax.axis_index("c")
        base = wid * rpw

        def body(i, carry):
            off = base + i * ch
            pltpu.sync_copy(idx_hbm.at[pl.ds(off, ch)], idx_v)
            pltpu.async_copy(table_hbm.at[idx_v], rows_v, sem).wait()
            pltpu.sync_copy(rows_v, out_hbm.at[pl.ds(off, ch)])
            return carry

        lax.fori_loop(0, nc, body, 0)

    return gk(table, idx)


def _sc_scatter(msg, dst_idx, n_rows):
    E, H = msg.shape
    per_core = E // _SC_CORES
    rpt = per_core // _SC_SUBCORES
    ch = _chunk_of(rpt)
    nc = rpt // ch
    np_rows = -(-n_rows // 3200) * 3200
    stripe = np_rows // _SC_SUBCORES
    zeros = jnp.zeros((np_rows, H), jnp.float32)

    @functools.partial(
        pl.kernel,
        out_type=jax.ShapeDtypeStruct((_SC_CORES, np_rows, H), jnp.float32),
        mesh=_sc_mesh(),
        scratch_types=[pltpu.VMEM((ch,), jnp.int32),
                       pltpu.VMEM((ch, H), jnp.float32),
                       pltpu.VMEM_SHARED((np_rows, H), jnp.float32),
                       pltpu.SemaphoreType.DMA],
    )
    def sk(msg_hbm, idx_hbm, zeros_hbm, out_hbm, idx_v, msg_v, acc_sh, sem):
        c = lax.axis_index("c")
        s = lax.axis_index("s")
        pltpu.sync_copy(zeros_hbm.at[pl.ds(s * stripe, stripe)],
                        acc_sh.at[pl.ds(s * stripe, stripe)])
        plsc.subcore_barrier()
        base = c * per_core + s * rpt

        def body(i, carry):
            off = base + i * ch
            pltpu.sync_copy(idx_hbm.at[pl.ds(off, ch)], idx_v)
            pltpu.sync_copy(msg_hbm.at[pl.ds(off, ch)], msg_v)
            pltpu.sync_copy(msg_v, acc_sh.at[idx_v], add=True)
            return carry

        lax.fori_loop(0, nc, body, 0)
        plsc.subcore_barrier()
        pltpu.sync_copy(acc_sh.at[pl.ds(s * stripe, stripe)],
                        out_hbm.at[c].at[pl.ds(s * stripe, stripe)])

    return sk(msg, dst_idx.astype(jnp.int32), zeros)


_PROBES_PER = 2500


def kernel(nodes, num_nodes, atom_edges, atom_edges_features, num_atom_edges,
           probe_edges, probe_edges_features, num_probes, num_probe_edges,
           params):
    bsz, nodes_per = nodes.shape
    N = bsz * nodes_per
    E = bsz * atom_edges.shape[1]

    idx_dtype = jnp.int32
    node_off = (jnp.arange(bsz, dtype=idx_dtype) * nodes_per)
    edges = (atom_edges.astype(idx_dtype)
             + node_off[:, None, None]).reshape(E, 2)
    e_src = edges[:, 0]
    e_dst = edges[:, 1]
    d_atom = atom_edges_features.reshape(E, 1).astype(jnp.float32)

    pe = bsz * probe_edges.shape[1]
    probes_per = _PROBES_PER
    P = bsz * probes_per
    probe_off = (jnp.arange(bsz, dtype=idx_dtype) * probes_per)
    pedges = probe_edges.astype(idx_dtype) + jnp.stack(
        [node_off, probe_off], axis=1)[:, None, :]
    pedges = pedges.reshape(pe, 2)
    pe_src = pedges[:, 0]
    pe_dst = pedges[:, 1]
    d_probe = probe_edges_features.reshape(pe, 1).astype(jnp.float32)

    h = _embed(nodes.reshape(N), params["atom_emb"])
    e_both = jnp.concatenate([e_src, e_dst])
    atom_reps = []
    for p in params["atom_int"]:
        g = _sc_gather(h, e_both)
        msg = _messages(E, g, 0, g, E, d_atom, p["msg"])
        parts = _sc_scatter(msg, e_dst, N)
        h = _atom_update(h, parts, p["st"])
        atom_reps.append(h)

    ps = jnp.zeros((P, h.shape[1]), dtype=jnp.float32)
    for i, (p, nod) in enumerate(zip(params["probe"], atom_reps)):
        hs = _sc_gather(nod, pe_src)
        if i == 0:
            msg = _messages(pe, hs, 0, None, 0, d_probe, p["msg"])
        else:
            hd = _sc_gather(ps, pe_dst)
            msg = _messages(pe, hs, 0, hd, 0, d_probe, p["msg"])
        parts = _sc_scatter(msg, pe_dst, P)
        ps = _probe_update(ps, parts, p["gate"], p["trans"])

    out = _readout(ps, params["readout"])
    return out.reshape(bsz, probes_per)


# speedup vs baseline: 2.9815x; 1.4328x over previous
import functools
import math

import jax
import jax.numpy as jnp
from jax import lax
from jax.experimental import pallas as pl
from jax.experimental.pallas import tpu as pltpu
from jax.experimental.pallas import tpu_sc as plsc

_SC_CORES = 2
_SC_SUBCORES = 16
_SC_WORKERS = _SC_CORES * _SC_SUBCORES


def _sc_mesh():
    return plsc.VectorSubcoreMesh(core_axis_name="c", subcore_axis_name="s",
                                  num_cores=_SC_CORES,
                                  num_subcores=_SC_SUBCORES)


def _chunk_of(rows):
    for c in range(128, 7, -8):
        if rows % c == 0:
            return c
    return rows

_CUTOFF = 4.0
_GSTEP = 0.1
_LN2 = math.log(2.0)


def _ssp(x):
    return jnp.maximum(x, 0.0) + jnp.log1p(jnp.exp(-jnp.abs(x))) - _LN2


def _sigmoid(x):
    return 1.0 / (1.0 + jnp.exp(-x))


def _pick_block(n, candidates):
    for c in candidates:
        if n % c == 0:
            return c
    return n


def _msg_body(with_dst, hs_ref, hd_ref, d_ref, w1a_ref, w1b_ref, b1_ref,
              w2_ref, b2_ref, we1_ref, be1_ref, we2_ref, be2_ref, out_ref):
    f32 = jnp.float32
    pre = jnp.dot(hs_ref[...], w1a_ref[...], preferred_element_type=f32)
    if with_dst:
        pre = pre + jnp.dot(hd_ref[...], w1b_ref[...],
                            preferred_element_type=f32)
    pre = pre + b1_ref[...][None, :]
    core = jnp.dot(_ssp(pre), w2_ref[...], preferred_element_type=f32)
    core = core + b2_ref[...][None, :]

    d = d_ref[...]
    nmu = we1_ref.shape[0]
    mu = _GSTEP * lax.broadcasted_iota(
        jnp.int32, (d.shape[0], nmu), 1).astype(f32)
    gauss = jnp.exp(-((d - mu) ** 2) / (2.0 * _GSTEP * _GSTEP))
    gh = _ssp(jnp.dot(gauss, we1_ref[...], preferred_element_type=f32)
              + be1_ref[...][None, :])
    gate = jnp.dot(gh, we2_ref[...], preferred_element_type=f32)
    gate = gate + be2_ref[...][None, :]
    cut = 1.0 - _sigmoid(5.0 * (d - (_CUTOFF - 1.5)))
    out_ref[...] = core * (gate * cut)


def _messages(E, hs, hs_base, hd, hd_base, d, mp):
    H = hs.shape[1]
    w1 = mp["node"]["w1"]
    w1a, w1b = w1[:H], w1[H:]
    eb = _pick_block(E, (2000, 1000, 800, 400, 200, 80, 40, 16, 8))
    grid = E // eb
    assert hs_base % eb == 0 and (hd_base % eb == 0)
    sb = hs_base // eb
    db = hd_base // eb
    with_dst = hd is not None
    if not with_dst:
        hd = hs

    def full(a):
        return pl.BlockSpec(a.shape, lambda i: (0,) * a.ndim)

    src_spec = pl.BlockSpec((eb, H), lambda i: (sb + i, 0))
    dst_spec = pl.BlockSpec((eb, H), lambda i: (db + i, 0))
    in_specs = [src_spec, dst_spec, pl.BlockSpec((eb, 1), lambda i: (i, 0)),
                full(w1a), full(w1b), full(mp["node"]["b1"]),
                full(mp["node"]["w2"]), full(mp["node"]["b2"]),
                full(mp["edge"]["w1"]), full(mp["edge"]["b1"]),
                full(mp["edge"]["w2"]), full(mp["edge"]["b2"])]
    out_h = mp["node"]["w2"].shape[1]
    return pl.pallas_call(
        functools.partial(_msg_body, with_dst),
        grid=(grid,),
        in_specs=in_specs,
        out_specs=pl.BlockSpec((eb, out_h), lambda i: (i, 0)),
        out_shape=jax.ShapeDtypeStruct((E, out_h), jnp.float32),
    )(hs, hd, d, w1a, w1b, mp["node"]["b1"], mp["node"]["w2"],
      mp["node"]["b2"], mp["edge"]["w1"], mp["edge"]["b1"],
      mp["edge"]["w2"], mp["edge"]["b2"])


def _atom_upd_body(h_ref, p0_ref, p1_ref, w1_ref, b1_ref, w2_ref, b2_ref,
                   out_ref):
    f32 = jnp.float32
    msum = p0_ref[0] + p1_ref[0]
    hid = _ssp(jnp.dot(msum, w1_ref[...], preferred_element_type=f32)
               + b1_ref[...][None, :])
    t = jnp.dot(hid, w2_ref[...], preferred_element_type=f32) \
        + b2_ref[...][None, :]
    out_ref[...] = h_ref[...] + t


def _atom_update(h, parts, sp):
    N, H = h.shape
    nb = 80

    def full(a):
        return pl.BlockSpec(a.shape, lambda i: (0,) * a.ndim)

    row = pl.BlockSpec((nb, H), lambda i: (i, 0))
    p0row = pl.BlockSpec((1, nb, H), lambda i: (0, i, 0))
    p1row = pl.BlockSpec((1, nb, H), lambda i: (1, i, 0))
    return pl.pallas_call(
        _atom_upd_body,
        grid=(N // nb,),
        in_specs=[row, p0row, p1row, full(sp["w1"]), full(sp["b1"]),
                  full(sp["w2"]), full(sp["b2"])],
        out_specs=row,
        out_shape=jax.ShapeDtypeStruct((N, H), jnp.float32),
    )(h, parts, parts, sp["w1"], sp["b1"], sp["w2"], sp["b2"])


def _probe_upd_body(ps_ref, p0_ref, p1_ref, gw1_ref, gb1_ref, gw2_ref,
                    gb2_ref, tw1_ref, tb1_ref, tw2_ref, tb2_ref, out_ref):
    f32 = jnp.float32
    ps = ps_ref[...]
    msum = p0_ref[0] + p1_ref[0]
    ghid = _ssp(jnp.dot(ps, gw1_ref[...], preferred_element_type=f32)
                + gb1_ref[...][None, :])
    g = _sigmoid(jnp.dot(ghid, gw2_ref[...], preferred_element_type=f32)
                 + gb2_ref[...][None, :])
    thid = _ssp(jnp.dot(msum, tw1_ref[...], preferred_element_type=f32)
                + tb1_ref[...][None, :])
    t = jnp.dot(thid, tw2_ref[...], preferred_element_type=f32) \
        + tb2_ref[...][None, :]
    out_ref[...] = ps * g + (1.0 - g) * t


def _probe_update(ps, parts, gp, tp):
    N, H = ps.shape
    nb = 80

    def full(a):
        return pl.BlockSpec(a.shape, lambda i: (0,) * a.ndim)

    row = pl.BlockSpec((nb, H), lambda i: (i, 0))
    p0row = pl.BlockSpec((1, nb, H), lambda i: (0, i, 0))
    p1row = pl.BlockSpec((1, nb, H), lambda i: (1, i, 0))
    return pl.pallas_call(
        _probe_upd_body,
        grid=(N // nb,),
        in_specs=[row, p0row, p1row,
                  full(gp["w1"]), full(gp["b1"]), full(gp["w2"]),
                  full(gp["b2"]),
                  full(tp["w1"]), full(tp["b1"]), full(tp["w2"]),
                  full(tp["b2"])],
        out_specs=row,
        out_shape=jax.ShapeDtypeStruct((N, H), jnp.float32),
    )(ps, parts, parts, gp["w1"], gp["b1"], gp["w2"], gp["b2"],
      tp["w1"], tp["b1"], tp["w2"], tp["b2"])


def _emb_body(idx_ref, emb_ref, out_ref):
    f32 = jnp.float32
    nel = emb_ref.shape[0]
    idx = idx_ref[...]
    iota = lax.broadcasted_iota(jnp.int32, (idx.shape[0], nel), 1)
    onehot = (idx == iota).astype(f32)
    out_ref[...] = jnp.dot(onehot, emb_ref[...], preferred_element_type=f32)


def _embed(nodes_flat, emb):
    N = nodes_flat.shape[0]
    H = emb.shape[1]
    nb = _pick_block(N, (2000, 1000, 800, 400, 200, 80, 40, 16, 8))
    idx2 = nodes_flat.reshape(N, 1).astype(jnp.int32)
    return pl.pallas_call(
        _emb_body,
        grid=(N // nb,),
        in_specs=[pl.BlockSpec((nb, 1), lambda i: (i, 0)),
                  pl.BlockSpec(emb.shape, lambda i: (0, 0))],
        out_specs=pl.BlockSpec((nb, H), lambda i: (i, 0)),
        out_shape=jax.ShapeDtypeStruct((N, H), jnp.float32),
    )(idx2, emb)


def _readout_body(ps_ref, w1_ref, b1_ref, w2_ref, b2_ref, out_ref):
    f32 = jnp.float32
    hid = _ssp(jnp.dot(ps_ref[...], w1_ref[...], preferred_element_type=f32)
               + b1_ref[...][None, :])
    out_ref[...] = jnp.dot(hid, w2_ref[...], preferred_element_type=f32) \
        + b2_ref[...][None, :]


def _readout(ps, rp):
    N, H = ps.shape
    nb = _pick_block(N, (2000, 1000, 800, 400, 200, 80, 40, 16, 8))

    def full(a):
        return pl.BlockSpec(a.shape, lambda i: (0,) * a.ndim)

    return pl.pallas_call(
        _readout_body,
        grid=(N // nb,),
        in_specs=[pl.BlockSpec((nb, H), lambda i: (i, 0)),
                  full(rp["w1"]), full(rp["b1"]), full(rp["w2"]),
                  full(rp["b2"])],
        out_specs=pl.BlockSpec((nb, 1), lambda i: (i, 0)),
        out_shape=jax.ShapeDtypeStruct((N, 1), jnp.float32),
    )(ps, rp["w1"], rp["b1"], rp["w2"], rp["b2"])


def _sc_gather(table, idx):
    E = idx.shape[0]
    H = table.shape[1]
    rpw = E // _SC_WORKERS
    ch = 128 if rpw >= 128 else _chunk_of(rpw)
    nc = rpw // ch
    tail = rpw - nc * ch
    assert tail % 8 == 0

    scratch = [pltpu.VMEM((rpw,), jnp.int32),
               pltpu.VMEM((ch, H), jnp.float32),
               pltpu.VMEM((ch, H), jnp.float32),
               pltpu.SemaphoreType.DMA,
               pltpu.SemaphoreType.DMA]
    if tail:
        scratch.append(pltpu.VMEM((tail, H), jnp.float32))

    @functools.partial(
        pl.kernel,
        out_type=jax.ShapeDtypeStruct((E, H), jnp.float32),
        mesh=_sc_mesh(),
        scratch_types=scratch,
    )
    def gk(table_hbm, idx_hbm, out_hbm, idx_v, rows0, rows1, sem0, sem1,
           *maybe_tail):
        wid = lax.axis_index("s") * _SC_CORES + lax.axis_index("c")
        base = wid * rpw
        pltpu.sync_copy(idx_hbm.at[pl.ds(base, rpw)], idx_v)
        rows = (rows0, rows1)
        sems = (sem0, sem1)

        def start(i, p):
            pltpu.async_copy(table_hbm.at[idx_v.at[pl.ds(i * ch, ch)]],
                             rows[p], sems[p])

        def wait(p):
            pltpu.make_async_copy(table_hbm.at[idx_v.at[pl.ds(0, ch)]],
                                  rows[p], sems[p]).wait()

        def wback(i, p):
            pltpu.sync_copy(rows[p], out_hbm.at[pl.ds(base + i * ch, ch)])

        start(0, 0)
        npairs = nc // 2

        def body(j, carry):
            i = 2 * j
            start(i + 1, 1)
            wait(0)
            wback(i, 0)

            @pl.when(i + 2 < nc)
            def _():
                start(i + 2, 0)

            wait(1)
            wback(i + 1, 1)
            return carry

        lax.fori_loop(0, npairs, body, 0)
        if nc % 2:
            wait(0)
            wback(nc - 1, 0)
        if tail:
            rows_t = maybe_tail[0]
            toff = base + nc * ch
            pltpu.async_copy(
                table_hbm.at[idx_v.at[pl.ds(nc * ch, tail)]],
                rows_t, sem0).wait()
            pltpu.sync_copy(rows_t, out_hbm.at[pl.ds(toff, tail)])

    return gk(table, idx)


def _sc_scatter(msg, dst_idx, n_rows):
    E, H = msg.shape
    per_core = E // _SC_CORES
    rpt = per_core // _SC_SUBCORES
    ch = 128 if rpt >= 128 else _chunk_of(rpt)
    nc = rpt // ch
    tail = rpt - nc * ch
    assert tail % 8 == 0
    np_rows = -(-n_rows // 640) * 640
    stripe = np_rows // _SC_SUBCORES
    zeros = jnp.zeros((np_rows, H), jnp.float32)

    scratch = [pltpu.VMEM((ch,), jnp.int32),
               pltpu.VMEM((ch,), jnp.int32),
               pltpu.VMEM((ch, H), jnp.float32),
               pltpu.VMEM((ch, H), jnp.float32),
               pltpu.VMEM_SHARED((np_rows, H), jnp.float32),
               pltpu.SemaphoreType.DMA,
               pltpu.SemaphoreType.DMA]
    if tail:
        scratch.extend([pltpu.VMEM((tail,), jnp.int32),
                        pltpu.VMEM((tail, H), jnp.float32)])

    @functools.partial(
        pl.kernel,
        out_type=jax.ShapeDtypeStruct((_SC_CORES, np_rows, H), jnp.float32),
        mesh=_sc_mesh(),
        scratch_types=scratch,
    )
    def sk(msg_hbm, idx_hbm, zeros_hbm, out_hbm, idx0, idx1, msg0, msg1,
           acc_sh, sem0, sem1, *maybe_tail):
        c = lax.axis_index("c")
        s = lax.axis_index("s")
        pltpu.sync_copy(zeros_hbm.at[pl.ds(s * stripe, stripe)],
                        acc_sh.at[pl.ds(s * stripe, stripe)])
        plsc.subcore_barrier()
        base = c * per_core + s * rpt
        idxb = (idx0, idx1)
        msgb = (msg0, msg1)
        sems = (sem0, sem1)

        def start(i, p):
            off = base + i * ch
            pltpu.async_copy(idx_hbm.at[pl.ds(off, ch)], idxb[p], sems[p])
            pltpu.async_copy(msg_hbm.at[pl.ds(off, ch)], msgb[p], sems[p])

        def wait(p):
            pltpu.make_async_copy(idx_hbm.at[pl.ds(0, ch)], idxb[p],
                                  sems[p]).wait()
            pltpu.make_async_copy(msg_hbm.at[pl.ds(0, ch)], msgb[p],
                                  sems[p]).wait()

        def scat(p):
            pltpu.sync_copy(msgb[p], acc_sh.at[idxb[p]], add=True)

        start(0, 0)
        npairs = nc // 2

        def body(j, carry):
            i = 2 * j
            start(i + 1, 1)
            wait(0)
            scat(0)

            @pl.when(i + 2 < nc)
            def _():
                start(i + 2, 0)

            wait(1)
            scat(1)
            return carry

        lax.fori_loop(0, npairs, body, 0)
        if nc % 2:
            wait(0)
            scat(0)
        if tail:
            idx_t, msg_t = maybe_tail
            toff = base + nc * ch
            pltpu.async_copy(idx_hbm.at[pl.ds(toff, tail)], idx_t, sem0)
            pltpu.async_copy(msg_hbm.at[pl.ds(toff, tail)], msg_t, sem0)
            pltpu.make_async_copy(idx_hbm.at[pl.ds(0, tail)], idx_t,
                                  sem0).wait()
            pltpu.make_async_copy(msg_hbm.at[pl.ds(0, tail)], msg_t,
                                  sem0).wait()
            pltpu.sync_copy(msg_t, acc_sh.at[idx_t], add=True)
        plsc.subcore_barrier()
        pltpu.sync_copy(acc_sh.at[pl.ds(s * stripe, stripe)],
                        out_hbm.at[c].at[pl.ds(s * stripe, stripe)])

    return sk(msg, dst_idx.astype(jnp.int32), zeros)


_PROBES_PER = 2500


def kernel(nodes, num_nodes, atom_edges, atom_edges_features, num_atom_edges,
           probe_edges, probe_edges_features, num_probes, num_probe_edges,
           params):
    bsz, nodes_per = nodes.shape
    N = bsz * nodes_per
    E = bsz * atom_edges.shape[1]

    idx_dtype = jnp.int32
    node_off = (jnp.arange(bsz, dtype=idx_dtype) * nodes_per)
    edges = (atom_edges.astype(idx_dtype)
             + node_off[:, None, None]).reshape(E, 2)
    e_src = edges[:, 0]
    e_dst = edges[:, 1]
    d_atom = atom_edges_features.reshape(E, 1).astype(jnp.float32)

    pe = bsz * probe_edges.shape[1]
    probes_per = _PROBES_PER
    P = bsz * probes_per
    probe_off = (jnp.arange(bsz, dtype=idx_dtype) * probes_per)
    pedges = probe_edges.astype(idx_dtype) + jnp.stack(
        [node_off, probe_off], axis=1)[:, None, :]
    pedges = pedges.reshape(pe, 2)
    pe_src = pedges[:, 0]
    pe_dst = pedges[:, 1]
    d_probe = probe_edges_features.reshape(pe, 1).astype(jnp.float32)

    h = _embed(nodes.reshape(N), params["atom_emb"])
    e_both = jnp.concatenate([e_src, e_dst])
    atom_reps = []
    for p in params["atom_int"]:
        g = _sc_gather(h, e_both)
        msg = _messages(E, g, 0, g, E, d_atom, p["msg"])
        parts = _sc_scatter(msg, e_dst, N)
        h = _atom_update(h, parts, p["st"])
        atom_reps.append(h)

    ps = jnp.zeros((P, h.shape[1]), dtype=jnp.float32)
    for i, (p, nod) in enumerate(zip(params["probe"], atom_reps)):
        hs = _sc_gather(nod, pe_src)
        if i == 0:
            msg = _messages(pe, hs, 0, None, 0, d_probe, p["msg"])
        else:
            hd = _sc_gather(ps, pe_dst)
            msg = _messages(pe, hs, 0, hd, 0, d_probe, p["msg"])
        parts = _sc_scatter(msg, pe_dst, P)
        ps = _probe_update(ps, parts, p["gate"], p["trans"])

    out = _readout(ps, params["readout"])
    return out.reshape(bsz, probes_per)


# speedup vs baseline: 3.0233x; 1.0140x over previous
import functools
import math

import jax
import jax.numpy as jnp
from jax import lax
from jax.experimental import pallas as pl
from jax.experimental.pallas import tpu as pltpu
from jax.experimental.pallas import tpu_sc as plsc

_SC_CORES = 2
_SC_SUBCORES = 16
_SC_WORKERS = _SC_CORES * _SC_SUBCORES


def _sc_mesh():
    return plsc.VectorSubcoreMesh(core_axis_name="c", subcore_axis_name="s",
                                  num_cores=_SC_CORES,
                                  num_subcores=_SC_SUBCORES)


def _chunk_of(rows):
    for c in range(128, 7, -8):
        if rows % c == 0:
            return c
    return rows

_CUTOFF = 4.0
_GSTEP = 0.1
_LN2 = math.log(2.0)


def _ssp(x):
    return jnp.maximum(x, 0.0) + jnp.log1p(jnp.exp(-jnp.abs(x))) - _LN2


def _sigmoid(x):
    return 1.0 / (1.0 + jnp.exp(-x))


def _pick_block(n, candidates):
    for c in candidates:
        if n % c == 0:
            return c
    return n


def _msg_body(with_dst, hs_ref, hd_ref, d_ref, w1a_ref, w1b_ref, b1_ref,
              w2_ref, b2_ref, we1_ref, be1_ref, we2_ref, be2_ref, out_ref):
    f32 = jnp.float32
    bf16 = jnp.bfloat16
    pre = jnp.dot(hs_ref[...].astype(bf16), w1a_ref[...].astype(bf16),
                  preferred_element_type=f32)
    if with_dst:
        pre = pre + jnp.dot(hd_ref[...].astype(bf16),
                            w1b_ref[...].astype(bf16),
                            preferred_element_type=f32)
    pre = pre + b1_ref[...][None, :]
    core = jnp.dot(_ssp(pre).astype(bf16), w2_ref[...].astype(bf16),
                   preferred_element_type=f32)
    core = core + b2_ref[...][None, :]

    d = d_ref[...]
    nmu = we1_ref.shape[0]
    mu = _GSTEP * lax.broadcasted_iota(
        jnp.int32, (d.shape[0], nmu), 1).astype(f32)
    gauss = jnp.exp(-((d - mu) ** 2) / (2.0 * _GSTEP * _GSTEP))
    gh = _ssp(jnp.dot(gauss.astype(bf16), we1_ref[...].astype(bf16),
                      preferred_element_type=f32)
              + be1_ref[...][None, :])
    gate = jnp.dot(gh.astype(bf16), we2_ref[...].astype(bf16),
                   preferred_element_type=f32)
    gate = gate + be2_ref[...][None, :]
    cut = 1.0 - _sigmoid(5.0 * (d - (_CUTOFF - 1.5)))
    out_ref[...] = core * (gate * cut)


def _messages(E, hs, hs_base, hd, hd_base, d, mp):
    H = hs.shape[1]
    w1 = mp["node"]["w1"]
    w1a, w1b = w1[:H], w1[H:]
    eb = _pick_block(E, (2000, 1000, 800, 400, 200, 80, 40, 16, 8))
    grid = E // eb
    assert hs_base % eb == 0 and (hd_base % eb == 0)
    sb = hs_base // eb
    db = hd_base // eb
    with_dst = hd is not None
    if not with_dst:
        hd = hs

    def full(a):
        return pl.BlockSpec(a.shape, lambda i: (0,) * a.ndim)

    src_spec = pl.BlockSpec((eb, H), lambda i: (sb + i, 0))
    dst_spec = pl.BlockSpec((eb, H), lambda i: (db + i, 0))
    in_specs = [src_spec, dst_spec, pl.BlockSpec((eb, 1), lambda i: (i, 0)),
                full(w1a), full(w1b), full(mp["node"]["b1"]),
                full(mp["node"]["w2"]), full(mp["node"]["b2"]),
                full(mp["edge"]["w1"]), full(mp["edge"]["b1"]),
                full(mp["edge"]["w2"]), full(mp["edge"]["b2"])]
    out_h = mp["node"]["w2"].shape[1]
    return pl.pallas_call(
        functools.partial(_msg_body, with_dst),
        grid=(grid,),
        in_specs=in_specs,
        out_specs=pl.BlockSpec((eb, out_h), lambda i: (i, 0)),
        out_shape=jax.ShapeDtypeStruct((E, out_h), jnp.float32),
    )(hs, hd, d, w1a, w1b, mp["node"]["b1"], mp["node"]["w2"],
      mp["node"]["b2"], mp["edge"]["w1"], mp["edge"]["b1"],
      mp["edge"]["w2"], mp["edge"]["b2"])


def _atom_upd_body(h_ref, p0_ref, p1_ref, w1_ref, b1_ref, w2_ref, b2_ref,
                   out_ref):
    f32 = jnp.float32
    msum = p0_ref[0] + p1_ref[0]
    hid = _ssp(jnp.dot(msum, w1_ref[...], preferred_element_type=f32)
               + b1_ref[...][None, :])
    t = jnp.dot(hid, w2_ref[...], preferred_element_type=f32) \
        + b2_ref[...][None, :]
    out_ref[...] = h_ref[...] + t


def _atom_update(h, parts, sp):
    N, H = h.shape
    nb = 80

    def full(a):
        return pl.BlockSpec(a.shape, lambda i: (0,) * a.ndim)

    row = pl.BlockSpec((nb, H), lambda i: (i, 0))
    p0row = pl.BlockSpec((1, nb, H), lambda i: (0, i, 0))
    p1row = pl.BlockSpec((1, nb, H), lambda i: (1, i, 0))
    return pl.pallas_call(
        _atom_upd_body,
        grid=(N // nb,),
        in_specs=[row, p0row, p1row, full(sp["w1"]), full(sp["b1"]),
                  full(sp["w2"]), full(sp["b2"])],
        out_specs=row,
        out_shape=jax.ShapeDtypeStruct((N, H), jnp.float32),
    )(h, parts, parts, sp["w1"], sp["b1"], sp["w2"], sp["b2"])


def _probe_upd_body(ps_ref, p0_ref, p1_ref, gw1_ref, gb1_ref, gw2_ref,
                    gb2_ref, tw1_ref, tb1_ref, tw2_ref, tb2_ref, out_ref):
    f32 = jnp.float32
    ps = ps_ref[...]
    msum = p0_ref[0] + p1_ref[0]
    ghid = _ssp(jnp.dot(ps, gw1_ref[...], preferred_element_type=f32)
                + gb1_ref[...][None, :])
    g = _sigmoid(jnp.dot(ghid, gw2_ref[...], preferred_element_type=f32)
                 + gb2_ref[...][None, :])
    thid = _ssp(jnp.dot(msum, tw1_ref[...], preferred_element_type=f32)
                + tb1_ref[...][None, :])
    t = jnp.dot(thid, tw2_ref[...], preferred_element_type=f32) \
        + tb2_ref[...][None, :]
    out_ref[...] = ps * g + (1.0 - g) * t


def _probe_update(ps, parts, gp, tp):
    N, H = ps.shape
    nb = 80

    def full(a):
        return pl.BlockSpec(a.shape, lambda i: (0,) * a.ndim)

    row = pl.BlockSpec((nb, H), lambda i: (i, 0))
    p0row = pl.BlockSpec((1, nb, H), lambda i: (0, i, 0))
    p1row = pl.BlockSpec((1, nb, H), lambda i: (1, i, 0))
    return pl.pallas_call(
        _probe_upd_body,
        grid=(N // nb,),
        in_specs=[row, p0row, p1row,
                  full(gp["w1"]), full(gp["b1"]), full(gp["w2"]),
                  full(gp["b2"]),
                  full(tp["w1"]), full(tp["b1"]), full(tp["w2"]),
                  full(tp["b2"])],
        out_specs=row,
        out_shape=jax.ShapeDtypeStruct((N, H), jnp.float32),
    )(ps, parts, parts, gp["w1"], gp["b1"], gp["w2"], gp["b2"],
      tp["w1"], tp["b1"], tp["w2"], tp["b2"])


def _emb_body(idx_ref, emb_ref, out_ref):
    f32 = jnp.float32
    nel = emb_ref.shape[0]
    idx = idx_ref[...]
    iota = lax.broadcasted_iota(jnp.int32, (idx.shape[0], nel), 1)
    onehot = (idx == iota).astype(f32)
    out_ref[...] = jnp.dot(onehot, emb_ref[...], preferred_element_type=f32)


def _embed(nodes_flat, emb):
    N = nodes_flat.shape[0]
    H = emb.shape[1]
    nb = _pick_block(N, (2000, 1000, 800, 400, 200, 80, 40, 16, 8))
    idx2 = nodes_flat.reshape(N, 1).astype(jnp.int32)
    return pl.pallas_call(
        _emb_body,
        grid=(N // nb,),
        in_specs=[pl.BlockSpec((nb, 1), lambda i: (i, 0)),
                  pl.BlockSpec(emb.shape, lambda i: (0, 0))],
        out_specs=pl.BlockSpec((nb, H), lambda i: (i, 0)),
        out_shape=jax.ShapeDtypeStruct((N, H), jnp.float32),
    )(idx2, emb)


def _readout_body(ps_ref, w1_ref, b1_ref, w2_ref, b2_ref, out_ref):
    f32 = jnp.float32
    hid = _ssp(jnp.dot(ps_ref[...], w1_ref[...], preferred_element_type=f32)
               + b1_ref[...][None, :])
    out_ref[...] = jnp.dot(hid, w2_ref[...], preferred_element_type=f32) \
        + b2_ref[...][None, :]


def _readout(ps, rp):
    N, H = ps.shape
    nb = _pick_block(N, (2000, 1000, 800, 400, 200, 80, 40, 16, 8))

    def full(a):
        return pl.BlockSpec(a.shape, lambda i: (0,) * a.ndim)

    return pl.pallas_call(
        _readout_body,
        grid=(N // nb,),
        in_specs=[pl.BlockSpec((nb, H), lambda i: (i, 0)),
                  full(rp["w1"]), full(rp["b1"]), full(rp["w2"]),
                  full(rp["b2"])],
        out_specs=pl.BlockSpec((nb, 1), lambda i: (i, 0)),
        out_shape=jax.ShapeDtypeStruct((N, 1), jnp.float32),
    )(ps, rp["w1"], rp["b1"], rp["w2"], rp["b2"])


def _sc_gather(table, idx):
    E = idx.shape[0]
    H = table.shape[1]
    rpw = E // _SC_WORKERS
    ch = 128 if rpw >= 128 else _chunk_of(rpw)
    nc = rpw // ch
    tail = rpw - nc * ch
    assert tail % 8 == 0

    scratch = [pltpu.VMEM((rpw,), jnp.int32),
               pltpu.VMEM((ch, H), jnp.float32),
               pltpu.VMEM((ch, H), jnp.float32),
               pltpu.SemaphoreType.DMA,
               pltpu.SemaphoreType.DMA]
    if tail:
        scratch.append(pltpu.VMEM((tail, H), jnp.float32))

    @functools.partial(
        pl.kernel,
        out_type=jax.ShapeDtypeStruct((E, H), jnp.float32),
        mesh=_sc_mesh(),
        scratch_types=scratch,
    )
    def gk(table_hbm, idx_hbm, out_hbm, idx_v, rows0, rows1, sem0, sem1,
           *maybe_tail):
        wid = lax.axis_index("s") * _SC_CORES + lax.axis_index("c")
        base = wid * rpw
        pltpu.sync_copy(idx_hbm.at[pl.ds(base, rpw)], idx_v)
        rows = (rows0, rows1)
        sems = (sem0, sem1)

        def start(i, p):
            pltpu.async_copy(table_hbm.at[idx_v.at[pl.ds(i * ch, ch)]],
                             rows[p], sems[p])

        def wait(p):
            pltpu.make_async_copy(table_hbm.at[idx_v.at[pl.ds(0, ch)]],
                                  rows[p], sems[p]).wait()

        def wback(i, p):
            pltpu.sync_copy(rows[p], out_hbm.at[pl.ds(base + i * ch, ch)])

        start(0, 0)
        npairs = nc // 2

        def body(j, carry):
            i = 2 * j
            start(i + 1, 1)
            wait(0)
            wback(i, 0)

            @pl.when(i + 2 < nc)
            def _():
                start(i + 2, 0)

            wait(1)
            wback(i + 1, 1)
            return carry

        lax.fori_loop(0, npairs, body, 0)
        if nc % 2:
            wait(0)
            wback(nc - 1, 0)
        if tail:
            rows_t = maybe_tail[0]
            toff = base + nc * ch
            pltpu.async_copy(
                table_hbm.at[idx_v.at[pl.ds(nc * ch, tail)]],
                rows_t, sem0).wait()
            pltpu.sync_copy(rows_t, out_hbm.at[pl.ds(toff, tail)])

    return gk(table, idx)


def _sc_scatter(msg, dst_idx, n_rows):
    E, H = msg.shape
    per_core = E // _SC_CORES
    rpt = per_core // _SC_SUBCORES
    ch = 128 if rpt >= 128 else _chunk_of(rpt)
    nc = rpt // ch
    tail = rpt - nc * ch
    assert tail % 8 == 0
    np_rows = -(-n_rows // 640) * 640
    stripe = np_rows // _SC_SUBCORES
    zeros = jnp.zeros((np_rows, H), jnp.float32)

    scratch = [pltpu.VMEM((ch,), jnp.int32),
               pltpu.VMEM((ch,), jnp.int32),
               pltpu.VMEM((ch, H), jnp.float32),
               pltpu.VMEM((ch, H), jnp.float32),
               pltpu.VMEM_SHARED((np_rows, H), jnp.float32),
               pltpu.SemaphoreType.DMA,
               pltpu.SemaphoreType.DMA]
    if tail:
        scratch.extend([pltpu.VMEM((tail,), jnp.int32),
                        pltpu.VMEM((tail, H), jnp.float32)])

    @functools.partial(
        pl.kernel,
        out_type=jax.ShapeDtypeStruct((_SC_CORES, np_rows, H), jnp.float32),
        mesh=_sc_mesh(),
        scratch_types=scratch,
    )
    def sk(msg_hbm, idx_hbm, zeros_hbm, out_hbm, idx0, idx1, msg0, msg1,
           acc_sh, sem0, sem1, *maybe_tail):
        c = lax.axis_index("c")
        s = lax.axis_index("s")
        pltpu.sync_copy(zeros_hbm.at[pl.ds(s * stripe, stripe)],
                        acc_sh.at[pl.ds(s * stripe, stripe)])
        plsc.subcore_barrier()
        base = c * per_core + s * rpt
        idxb = (idx0, idx1)
        msgb = (msg0, msg1)
        sems = (sem0, sem1)

        def start(i, p):
            off = base + i * ch
            pltpu.async_copy(idx_hbm.at[pl.ds(off, ch)], idxb[p], sems[p])
            pltpu.async_copy(msg_hbm.at[pl.ds(off, ch)], msgb[p], sems[p])

        def wait(p):
            pltpu.make_async_copy(idx_hbm.at[pl.ds(0, ch)], idxb[p],
                                  sems[p]).wait()
            pltpu.make_async_copy(msg_hbm.at[pl.ds(0, ch)], msgb[p],
                                  sems[p]).wait()

        def scat(p):
            pltpu.sync_copy(msgb[p], acc_sh.at[idxb[p]], add=True)

        start(0, 0)
        npairs = nc // 2

        def body(j, carry):
            i = 2 * j
            start(i + 1, 1)
            wait(0)
            scat(0)

            @pl.when(i + 2 < nc)
            def _():
                start(i + 2, 0)

            wait(1)
            scat(1)
            return carry

        lax.fori_loop(0, npairs, body, 0)
        if nc % 2:
            wait(0)
            scat(0)
        if tail:
            idx_t, msg_t = maybe_tail
            toff = base + nc * ch
            pltpu.async_copy(idx_hbm.at[pl.ds(toff, tail)], idx_t, sem0)
            pltpu.async_copy(msg_hbm.at[pl.ds(toff, tail)], msg_t, sem0)
            pltpu.make_async_copy(idx_hbm.at[pl.ds(0, tail)], idx_t,
                                  sem0).wait()
            pltpu.make_async_copy(msg_hbm.at[pl.ds(0, tail)], msg_t,
                                  sem0).wait()
            pltpu.sync_copy(msg_t, acc_sh.at[idx_t], add=True)
        plsc.subcore_barrier()
        pltpu.sync_copy(acc_sh.at[pl.ds(s * stripe, stripe)],
                        out_hbm.at[c].at[pl.ds(s * stripe, stripe)])

    return sk(msg, dst_idx.astype(jnp.int32), zeros)


_PROBES_PER = 2500


def kernel(nodes, num_nodes, atom_edges, atom_edges_features, num_atom_edges,
           probe_edges, probe_edges_features, num_probes, num_probe_edges,
           params):
    bsz, nodes_per = nodes.shape
    N = bsz * nodes_per
    E = bsz * atom_edges.shape[1]

    idx_dtype = jnp.int32
    node_off = (jnp.arange(bsz, dtype=idx_dtype) * nodes_per)
    edges = (atom_edges.astype(idx_dtype)
             + node_off[:, None, None]).reshape(E, 2)
    e_src = edges[:, 0]
    e_dst = edges[:, 1]
    d_atom = atom_edges_features.reshape(E, 1).astype(jnp.float32)

    pe = bsz * probe_edges.shape[1]
    probes_per = _PROBES_PER
    P = bsz * probes_per
    probe_off = (jnp.arange(bsz, dtype=idx_dtype) * probes_per)
    pedges = probe_edges.astype(idx_dtype) + jnp.stack(
        [node_off, probe_off], axis=1)[:, None, :]
    pedges = pedges.reshape(pe, 2)
    pe_src = pedges[:, 0]
    pe_dst = pedges[:, 1]
    d_probe = probe_edges_features.reshape(pe, 1).astype(jnp.float32)

    h = _embed(nodes.reshape(N), params["atom_emb"])
    e_both = jnp.concatenate([e_src, e_dst])
    atom_reps = []
    for p in params["atom_int"]:
        g = _sc_gather(h, e_both)
        msg = _messages(E, g, 0, g, E, d_atom, p["msg"])
        parts = _sc_scatter(msg, e_dst, N)
        h = _atom_update(h, parts, p["st"])
        atom_reps.append(h)

    ps = jnp.zeros((P, h.shape[1]), dtype=jnp.float32)
    for i, (p, nod) in enumerate(zip(params["probe"], atom_reps)):
        hs = _sc_gather(nod, pe_src)
        if i == 0:
            msg = _messages(pe, hs, 0, None, 0, d_probe, p["msg"])
        else:
            hd = _sc_gather(ps, pe_dst)
            msg = _messages(pe, hs, 0, hd, 0, d_probe, p["msg"])
        parts = _sc_scatter(msg, pe_dst, P)
        ps = _probe_update(ps, parts, p["gate"], p["trans"])

    out = _readout(ps, params["readout"])
    return out.reshape(bsz, probes_per)
